# baseline (speedup 1.0000x reference)
import functools

import jax
import jax.numpy as jnp
from jax import lax
from jax.experimental import pallas as pl
from jax.experimental.pallas import tpu as pltpu
from jax.experimental.pallas import tpu_sc as plsc

NODE_DIM = 128
OUT_DIM = 128
NUM_HEADS = 8
HEAD_DIM = 16
HID = NUM_HEADS * HEAD_DIM
N_NODES = 100000
NUM_GRAPHS = 1024
BLK = 1024
NBLK = 98
N_PAD = NBLK * BLK

NEG = -3.0e38

PAYC = 400
NTILES = 32
ROWS_PER_TILE = N_PAD // NTILES
CHUNK = 224
NCHUNK = ROWS_PER_TILE // CHUNK
EDGE_BASE = NUM_GRAPHS
TRASH = NUM_GRAPHS + 64
OUTROWS = NUM_GRAPHS + 80
NOMATCH = 4000


def _payload_body(x_ref, idxc_ref,
                  mWs1, mbs1, mWs2, mbs2, mWt1, mbt1, mWt2, mbt2,
                  sWs1, sbs1, sWs2, sbs2, sWt1, sbt1, sWt2, sbt2,
                  pay_ref):
    blk = x_ref.shape[0]
    pid = pl.program_id(0)
    x = x_ref[...]

    rows = pid * blk + lax.broadcasted_iota(jnp.int32, (blk, 1), 0)
    validf = (rows < N_NODES).astype(jnp.float32)

    eh = lax.broadcasted_iota(jnp.int32, (NUM_HEADS, HID), 0)
    ej = lax.broadcasted_iota(jnp.int32, (NUM_HEADS, HID), 1) // HEAD_DIM
    E = (eh == ej).astype(jnp.float32)

    def mlp(W1, b1, W2, b2):
        h = jnp.maximum(
            jnp.dot(x, W1[...], preferred_element_type=jnp.float32) + b1[...], 0.0)
        return jnp.dot(h, W2[...], preferred_element_type=jnp.float32) + b2[...]

    s_m = mlp(mWs1, mbs1, mWs2, mbs2)
    e_m = jnp.exp(s_m) * validf
    v_m = mlp(mWt1, mbt1, mWt2, mbt2)
    wv_m = jnp.dot(e_m, E, preferred_element_type=jnp.float32) * v_m

    s_s = mlp(sWs1, sbs1, sWs2, sbs2)
    w_s = validf / (1.0 + jnp.exp(-s_s))
    v_s = mlp(sWt1, sbt1, sWt2, sbt2)
    wv_s = jnp.dot(w_s, E, preferred_element_type=jnp.float32) * v_s

    xm = jnp.where(validf > 0.0, x, NEG)
    pay_ref[...] = jnp.concatenate(
        [wv_m, wv_s, xm, e_m, jnp.zeros((blk, 8), jnp.float32)], axis=1)


def _tc_payload(x, idxc3, weights):
    (mWs1, mbs1, mWs2, mbs2, mWt1, mbt1, mWt2, mbt2,
     sWs1, sbs1, sWs2, sbs2, sWt1, sbt1, sWt2, sbt2) = weights
    full = lambda shp: pl.BlockSpec(shp, lambda i: tuple(0 for _ in shp))
    in_specs = [
        pl.BlockSpec((BLK, NODE_DIM), lambda i: (i, 0)),
        pl.BlockSpec((1, BLK, 1), lambda i: (i, 0, 0)),
        full((NODE_DIM, HID)), full((1, HID)), full((HID, NUM_HEADS)), full((1, NUM_HEADS)),
        full((NODE_DIM, HID)), full((1, HID)), full((HID, HID)), full((1, HID)),
        full((NODE_DIM, HID)), full((1, HID)), full((HID, NUM_HEADS)), full((1, NUM_HEADS)),
        full((NODE_DIM, HID)), full((1, HID)), full((HID, HID)), full((1, HID)),
    ]
    return pl.pallas_call(
        _payload_body,
        grid=(NBLK,),
        in_specs=in_specs,
        out_specs=pl.BlockSpec((BLK, PAYC), lambda i: (i, 0)),
        out_shape=jax.ShapeDtypeStruct((N_PAD, PAYC), jnp.float32),
        compiler_params=pltpu.CompilerParams(
            dimension_semantics=("arbitrary",),
        ),
    )(x, idxc3,
      mWs1, mbs1, mWs2, mbs2, mWt1, mbt1, mWt2, mbt2,
      sWs1, sbs1, sWs2, sbs2, sWt1, sbt1, sWt2, sbt2)


def _sc_segreduce(pay1d, idxe, ob1, ob2, ob3, ob4):
    mesh = plsc.VectorSubcoreMesh(core_axis_name="c", subcore_axis_name="s")

    @functools.partial(
        pl.kernel,
        out_type=(),
        mesh=mesh,
        scratch_types=[
            pltpu.VMEM((CHUNK * PAYC,), jnp.float32),
            pltpu.VMEM((CHUNK + 16,), jnp.int32),
            pltpu.VMEM((16, 128), jnp.float32),
            pltpu.VMEM((16, 128), jnp.float32),
            pltpu.VMEM((16, 128), jnp.float32),
            pltpu.VMEM((16, 128), jnp.float32),
            pltpu.VMEM((16,), jnp.int32),
        ],
    )
    def k(pay_hbm, idxe_hbm, o1, o2, o3, o4, pbuf, ibuf, f1, f2, f3, f4, sidx):
        c = lax.axis_index("c")
        s = lax.axis_index("s")
        wid = s * 2 + c
        iota16 = lax.iota(jnp.int32, 16)
        zero16 = jnp.zeros((16,), jnp.float32)
        neg16 = jnp.full((16,), NEG, jnp.float32)
        base0 = wid * ROWS_PER_TILE

        def flush(dest, gid, sumv, maxv, ev):
            for j in range(8):
                f1[0, pl.ds(j * 16, 16)] = sumv[j]
                f2[0, pl.ds(j * 16, 16)] = sumv[8 + j]
                f3[0, pl.ds(j * 16, 16)] = maxv[j]
            evg = jnp.where(iota16 == 8, gid.astype(jnp.float32), ev)
            f4[0, pl.ds(0, 16)] = evg
            sidx[...] = jnp.where(iota16 == 0, dest, TRASH)
            pltpu.sync_copy(f1, o1.at[sidx])
            pltpu.sync_copy(f2, o2.at[sidx])
            pltpu.sync_copy(f3, o3.at[sidx])
            pltpu.sync_copy(f4, o4.at[sidx])

        def chunk_body(ci, carry):
            rbase = pl.multiple_of(base0 + ci * CHUNK, 8)
            pltpu.sync_copy(pay_hbm.at[pl.ds(rbase * PAYC, CHUNK * PAYC)], pbuf)
            pltpu.sync_copy(idxe_hbm.at[pl.ds(rbase, CHUNK + 16)], ibuf)

            def row_body(i, carry):
                sumv, maxv, ev, nfl = carry
                off = i * PAYC
                sumv = tuple(sumv[j] + pbuf[pl.ds(off + j * 16, 16)]
                             for j in range(16))
                maxv = tuple(
                    jnp.maximum(maxv[j], pbuf[pl.ds(off + 256 + j * 16, 16)])
                    for j in range(8))
                ev = ev + pbuf[pl.ds(off + 384, 16)]
                iv = ibuf[pl.ds(i, 16)]
                gid = iv[0]
                nxt = iv[1]
                isb = gid != nxt

                def do_flush(a):
                    sv, mv, e1 = a
                    dest = jnp.where(nfl == 0, EDGE_BASE + 2 * wid, gid)
                    flush(dest, gid, sv, mv, e1)
                    return (tuple(zero16 for _ in range(16)),
                            tuple(neg16 for _ in range(8)),
                            zero16)

                sumv, maxv, ev = lax.cond(
                    isb, do_flush, lambda a: a, (sumv, maxv, ev))
                nfl = jnp.where(isb, nfl + 1, nfl)
                return (sumv, maxv, ev, nfl)

            return lax.fori_loop(0, CHUNK, row_body, carry)

        carry0 = (tuple(zero16 for _ in range(16)),
                  tuple(neg16 for _ in range(8)),
                  zero16, jnp.int32(0))
        sumv, maxv, ev, nfl = lax.fori_loop(0, NCHUNK, chunk_body, carry0)
        lastv = ibuf[pl.ds(CHUNK - 1, 16)]
        flush(EDGE_BASE + 2 * wid + 1, lastv[0], sumv, maxv, ev)

    return k(pay1d, idxe, ob1, ob2, ob3, ob4)


def _final_body(m1, m2, m3, m4, e1, e2, e3, e4, egr,
                mWc, sWc, maxWc, combW, out_ref):
    gidrow = egr[...]

    gcol = lax.broadcasted_iota(jnp.int32, (NUM_GRAPHS, 64), 0)
    EOT = (gcol == gidrow.astype(jnp.int32)).astype(jnp.float32)

    wvm = m1[...] + jnp.dot(EOT, e1[...], preferred_element_type=jnp.float32)
    wvs = m2[...] + jnp.dot(EOT, e2[...], preferred_element_type=jnp.float32)
    z16 = m4[...] + jnp.dot(EOT, e4[...], preferred_element_type=jnp.float32)
    z = z16[:, 0:8]

    am = m3[...]
    e3v = e3[...]
    for e in range(64):
        vm = EOT[:, e:e + 1] > 0.5
        val = e3v[e:e + 1, :]
        am = jnp.maximum(am, jnp.where(vm, val, NEG))

    eh = lax.broadcasted_iota(jnp.int32, (NUM_HEADS, HID), 0)
    ej = lax.broadcasted_iota(jnp.int32, (NUM_HEADS, HID), 1) // HEAD_DIM
    E = (eh == ej).astype(jnp.float32)

    zinv = 1.0 / jnp.where(z == 0.0, 1.0, z)
    mean_pre = wvm * jnp.dot(zinv, E, preferred_element_type=jnp.float32)
    maxv = jnp.where(am <= -1.0e38, 0.0, am)
    mean_repr = jnp.dot(mean_pre, mWc[...], preferred_element_type=jnp.float32)
    sum_repr = jnp.dot(wvs, sWc[...], preferred_element_type=jnp.float32)
    max_repr = jnp.dot(maxv, maxWc[...], preferred_element_type=jnp.float32)
    cw = combW[...]
    out_ref[...] = (
        jnp.dot(mean_repr, cw[0:128, :], preferred_element_type=jnp.float32)
        + jnp.dot(sum_repr, cw[128:256, :], preferred_element_type=jnp.float32)
        + jnp.dot(max_repr, cw[256:384, :], preferred_element_type=jnp.float32))


def _tc_final(m1, m2, m3, m4, e1, e2, e3, e4, egr, mWc, sWc, maxWc, combW):
    full = lambda shp: pl.BlockSpec(shp, lambda: tuple(0 for _ in shp))
    return pl.pallas_call(
        _final_body,
        in_specs=[
            full((NUM_GRAPHS, 128)), full((NUM_GRAPHS, 128)),
            full((NUM_GRAPHS, 128)), full((NUM_GRAPHS, 128)),
            full((64, 128)), full((64, 128)), full((64, 128)), full((64, 128)),
            full((1, 64)),
            full((HID, OUT_DIM)), full((HID, OUT_DIM)),
            full((NODE_DIM, OUT_DIM)), full((3 * OUT_DIM, OUT_DIM)),
        ],
        out_specs=full((NUM_GRAPHS, OUT_DIM)),
        out_shape=jax.ShapeDtypeStruct((NUM_GRAPHS, OUT_DIM), jnp.float32),
    )(m1, m2, m3, m4, e1, e2, e3, e4, egr, mWc, sWc, maxWc, combW)


def kernel(node_embeddings, node_to_graph_id, num_graphs,
           mean_Ws1, mean_bs1, mean_Ws2, mean_bs2, mean_Wt1, mean_bt1,
           mean_Wt2, mean_bt2, mean_Wc,
           sum_Ws1, sum_bs1, sum_Ws2, sum_bs2, sum_Wt1, sum_bt1,
           sum_Wt2, sum_bt2, sum_Wc,
           max_Wc, comb_W):
    del num_graphs
    x = jnp.pad(node_embeddings, ((0, N_PAD - N_NODES), (0, 0)))
    idx = jnp.pad(node_to_graph_id.astype(jnp.int32), (0, N_PAD - N_NODES),
                  constant_values=NUM_GRAPHS - 1)
    idxc3 = idx.reshape(NBLK, BLK, 1)
    idxe = jnp.pad(idx, (0, 16), constant_values=2 * NUM_GRAPHS)

    b2 = lambda b: b.reshape(1, -1)
    pay = _tc_payload(
        x, idxc3,
        (mean_Ws1, b2(mean_bs1), mean_Ws2, b2(mean_bs2),
         mean_Wt1, b2(mean_bt1), mean_Wt2, b2(mean_bt2),
         sum_Ws1, b2(sum_bs1), sum_Ws2, b2(sum_bs2),
         sum_Wt1, b2(sum_bt1), sum_Wt2, b2(sum_bt2)))

    zinit = jnp.zeros((OUTROWS, 128), jnp.float32)
    ob1 = jax.new_ref(zinit)
    ob2 = jax.new_ref(zinit)
    ob3 = jax.new_ref(jnp.full((OUTROWS, 128), NEG, jnp.float32))
    init4 = jnp.concatenate([
        jnp.zeros((OUTROWS, 8), jnp.float32),
        jnp.full((OUTROWS, 1), float(NOMATCH), jnp.float32),
        jnp.zeros((OUTROWS, 119), jnp.float32)], axis=1)
    ob4 = jax.new_ref(init4)
    _sc_segreduce(pay.reshape(-1), idxe, ob1, ob2, ob3, ob4)
    r1, r2, r3, r4 = ob1[...], ob2[...], ob3[...], ob4[...]

    eb = EDGE_BASE
    egr = r4[eb:eb + 64, 8:9].reshape(1, 64)
    return _tc_final(
        r1[0:NUM_GRAPHS], r2[0:NUM_GRAPHS], r3[0:NUM_GRAPHS], r4[0:NUM_GRAPHS],
        r1[eb:eb + 64], r2[eb:eb + 64], r3[eb:eb + 64], r4[eb:eb + 64],
        egr, mean_Wc, sum_Wc, max_Wc, comb_W)
